# baseline (speedup 1.0000x reference)
import functools

import jax
import jax.numpy as jnp
from jax import lax
from jax.experimental import pallas as pl
from jax.experimental.pallas import tpu as pltpu
from jax.experimental.pallas import tpu_sc as plsc

C = 16
GX = GY = GZ = 128
V = GX * GY * GZ
N = 1048576


_IDX_ROWS = N // 128
_IDX_BR = 512


def _idx_body(c_ref, o_ref):
    x = c_ref[0]
    y = c_ref[1]
    z = c_ref[2]
    ix = jnp.round(x * (GX - 1.0)).astype(jnp.int32)
    iy = jnp.round(y * (GY - 1.0)).astype(jnp.int32)
    iz = jnp.round(z * (GZ - 1.0)).astype(jnp.int32)
    ix = jnp.clip(ix, 0, GX - 1)
    iy = jnp.clip(iy, 0, GY - 1)
    iz = jnp.clip(iz, 0, GZ - 1)
    o_ref[...] = (ix * GY + iy) * GZ + iz


def _compute_indices(coords_t):
    return pl.pallas_call(
        _idx_body,
        grid=(_IDX_ROWS // _IDX_BR,),
        in_specs=[pl.BlockSpec((3, _IDX_BR, 128), lambda i: (0, i, 0))],
        out_specs=pl.BlockSpec((_IDX_BR, 128), lambda i: (i, 0)),
        out_shape=jax.ShapeDtypeStruct((_IDX_ROWS, 128), jnp.int32),
    )(coords_t)


_NC = 2
_NS = 16
_NW = _NC * _NS
_BPW = N // _NW
_GRP = 8
_IPG = 128
_RPG = _GRP * _IPG
_NGRP = _BPW // _RPG


def _gather_kernel(idx, table):
    mesh = plsc.VectorSubcoreMesh(core_axis_name="c", subcore_axis_name="s")

    @functools.partial(
        pl.kernel,
        mesh=mesh,
        compiler_params=pltpu.CompilerParams(use_tc_tiling_on_sc=False),
        out_type=jax.ShapeDtypeStruct((N, C), jnp.float32),
        scratch_types=[
            pltpu.VMEM((_GRP, _IPG), jnp.int32),
            pltpu.VMEM((_RPG, C), jnp.float32),
            pltpu.SemaphoreType.DMA,
        ],
    )
    def body(idx_hbm, table_hbm, out_hbm, idx_v, rows_v, sem):
        wid = lax.axis_index("s") * _NC + lax.axis_index("c")
        base = wid * _BPW

        def group(g, carry):
            off = base + g * _RPG
            row0 = pl.multiple_of(off // _IPG, _GRP)
            pltpu.sync_copy(idx_hbm.at[pl.ds(row0, _GRP)], idx_v)
            copies = []
            for j in range(_GRP):
                copies.append(pltpu.async_copy(
                    table_hbm.at[idx_v.at[j]],
                    rows_v.at[pl.ds(j * _IPG, _IPG)],
                    sem))
            for cp in copies:
                cp.wait()
            pltpu.sync_copy(rows_v, out_hbm.at[pl.ds(off, _RPG)])
            return carry

        lax.fori_loop(0, _NGRP, group, 0)

    return body(idx, table)


def kernel(coords, grid):
    table = grid.reshape(C, V).T
    coords_t = coords.T.reshape(3, _IDX_ROWS, 128)
    idx = _compute_indices(coords_t)
    idx2d = idx.reshape(N // _IPG, _IPG)
    return _gather_kernel(idx2d, table)
